# baseline (speedup 1.0000x reference)
import functools

import jax
import jax.numpy as jnp
from jax import lax
from jax.experimental import pallas as pl
from jax.experimental.pallas import tpu as pltpu
from jax.experimental.pallas import tpu_sc as plsc

NC = 2
NS = 16
NW = NC * NS
LANES = 16
BLK = 2 * LANES
CHUNK = 128
NBUF = 4


def _tc_pre_body(x_ref, we_ref, y_ref, *, n, n_pad, d_hid):
    y = jnp.dot(x_ref[...], we_ref[...].T, preferred_element_type=jnp.float32)
    y_ref[:n, :] = y
    y_ref[n:, :] = jnp.zeros((n_pad - n, d_hid), jnp.float32)


def _tc_post_body(x_ref, p_ref, wn_ref, b2_ref, wm_ref, bm_ref, o_ref, *, n):
    m = (
        jnp.dot(x_ref[...], wn_ref[...].T, preferred_element_type=jnp.float32)
        + b2_ref[...]
        + p_ref[0, :n, :]
        + p_ref[1, :n, :]
    )
    o_ref[...] = jnp.dot(m, wm_ref[...].T, preferred_element_type=jnp.float32) + bm_ref[...]


def _sc_segment_sum(y_hbm, src_hbm, dst_hbm, w_hbm, out_hbm,
                    srcv, dstv, wv, gbufs, sbufs, accsp, gsems,
                    *, nch, n_pad, d_hid, rows_per_sub):
    cid = lax.axis_index("c")
    sid = lax.axis_index("s")
    wid = cid * NS + sid
    base = sid * rows_per_sub

    @pl.loop(0, CHUNK)
    def _(i):
        for t in range(d_hid // LANES):
            sbufs[0][i, pl.ds(t * LANES, LANES)] = jnp.zeros((LANES,), jnp.float32)

    @pl.loop(0, rows_per_sub // CHUNK)
    def _(k):
        pltpu.sync_copy(sbufs[0], accsp.at[pl.ds(base + k * CHUNK, CHUNK)])

    pltpu.sync_copy(src_hbm.at[wid], srcv)
    pltpu.sync_copy(dst_hbm.at[wid], dstv)
    pltpu.sync_copy(w_hbm.at[wid], wv)

    plsc.subcore_barrier()

    def gather(j, buf, sem):
        pltpu.async_copy(y_hbm.at[srcv.at[j]], buf, sem)

    def gather_wait(j, buf, sem):
        pltpu.make_async_copy(y_hbm.at[srcv.at[j]], buf, sem).wait()

    def mul(j, gbuf, sbuf):
        @pl.loop(0, CHUNK // LANES)
        def _(g):
            wvec = wv[j, pl.ds(g * LANES, LANES)]
            for q in range(LANES):
                w16 = jnp.full((LANES,), wvec[q], jnp.float32)
                e = g * LANES + q
                for t in range(d_hid // BLK):
                    row = gbuf[e, pl.ds(t * BLK, BLK)]
                    lo, hi = plsc.unpack(row, format=plsc.PackFormat.INTERLEAVED,
                                         preferred_element_type=jnp.float32)
                    sbuf[e, pl.ds(t * BLK, LANES)] = lo * w16
                    sbuf[e, pl.ds(t * BLK + LANES, LANES)] = hi * w16

    for b in range(NBUF):
        gather(b, gbufs[b], gsems[b])

    @pl.loop(0, nch // NBUF)
    def _(k):
        for b in range(NBUF):
            j = k * NBUF + b
            bprev = (b - 2) % NBUF

            @pl.when(jnp.logical_and(j >= 2, j + 2 < nch))
            def _():
                gather(j + 2, gbufs[bprev], gsems[bprev])

            gather_wait(j, gbufs[b], gsems[b])
            mul(j, gbufs[b], sbufs[b])
            pltpu.sync_copy(sbufs[b], accsp.at[dstv.at[j]], add=True)

    plsc.subcore_barrier()

    pltpu.sync_copy(accsp.at[pl.ds(base, rows_per_sub)],
                    out_hbm.at[pl.ds(cid * n_pad + base, rows_per_sub)])


def kernel(x, edge_index, edge_weight, W_node, b_node, W_edge, b_edge, W_mlp, b_mlp):
    n, d_in = x.shape
    e = edge_weight.shape[0]
    d_hid = W_node.shape[0]
    d_out = W_mlp.shape[0]

    epw = -(-e // NW)
    nch = -(-epw // CHUNK)
    nch = -(-nch // NBUF) * NBUF
    e_pad = NW * nch * CHUNK
    n_pad = -(-n // (NS * 8)) * (NS * 8)
    rows_per_sub = n_pad // NS

    src = jnp.pad(edge_index[0], (0, e_pad - e)).reshape(NW, nch, CHUNK)
    dst = jnp.pad(edge_index[1], (0, e_pad - e)).reshape(NW, nch, CHUNK)
    w = jnp.pad(edge_weight, (0, e_pad - e)).reshape(NW, nch, CHUNK)

    perm = [blk * BLK + (i % 2) * LANES + i // 2
            for blk in range(d_hid // BLK) for i in range(BLK)]
    y = pl.pallas_call(
        functools.partial(_tc_pre_body, n=n, n_pad=n_pad, d_hid=d_hid),
        out_shape=jax.ShapeDtypeStruct((n_pad, d_hid), jnp.float32),
    )(x, W_edge[jnp.array(perm), :]).astype(jnp.bfloat16)

    sc = functools.partial(
        pl.kernel,
        out_type=jax.ShapeDtypeStruct((NC * n_pad, d_hid), jnp.float32),
        mesh=plsc.VectorSubcoreMesh(core_axis_name="c", subcore_axis_name="s"),
        scratch_types=[
            pltpu.VMEM((nch, CHUNK), jnp.int32),
            pltpu.VMEM((nch, CHUNK), jnp.int32),
            pltpu.VMEM((nch, CHUNK), jnp.float32),
            [pltpu.VMEM((CHUNK, d_hid), jnp.bfloat16) for _ in range(NBUF)],
            [pltpu.VMEM((CHUNK, d_hid), jnp.float32) for _ in range(NBUF)],
            pltpu.VMEM_SHARED((n_pad, d_hid), jnp.float32),
            [pltpu.SemaphoreType.DMA for _ in range(NBUF)],
        ],
        compiler_params=pltpu.CompilerParams(use_tc_tiling_on_sc=False,
                                             needs_layout_passes=False),
    )(functools.partial(_sc_segment_sum, nch=nch, n_pad=n_pad, d_hid=d_hid,
                        rows_per_sub=rows_per_sub))
    partials = sc(y, src, dst, w).reshape(NC, n_pad, d_hid)

    b2 = (b_node + b_edge).reshape(1, d_hid)
    out = pl.pallas_call(
        functools.partial(_tc_post_body, n=n),
        out_shape=jax.ShapeDtypeStruct((n, d_out), jnp.float32),
    )(x, partials, W_node, b2, W_mlp, b_mlp.reshape(1, d_out))
    return out
